# Initial kernel scaffold; baseline (speedup 1.0000x reference)
import jax
import jax.numpy as jnp
from jax.experimental import pallas as pl


def kernel(x, edge_index, W1, b1, W2, b2):
    raise NotImplementedError("write your pallas kernel here")


# speedup vs baseline: 9.0260x; 9.0260x over previous
import functools

import jax
import jax.numpy as jnp
from jax import lax
from jax.experimental import pallas as pl
from jax.experimental.pallas import tpu as pltpu
from jax.experimental.pallas import tpu_sc as plsc

_NSUB = 16
_NCORE = 2
_CHUNK = 80


def _make_deg(npad, e):
    half = e // _NCORE
    epw = half // _NSUB
    iters = epw // _CHUNK
    npw = npad // _NSUB
    mesh = plsc.VectorSubcoreMesh(core_axis_name="c", subcore_axis_name="s")

    @functools.partial(
        pl.kernel,
        mesh=mesh,
        compiler_params=pltpu.CompilerParams(use_tc_tiling_on_sc=False),
        out_type=jax.ShapeDtypeStruct((_NCORE * npad, 16), jnp.float32),
        scratch_types=[
            pltpu.VMEM((_CHUNK,), jnp.int32),
            pltpu.VMEM((_CHUNK, 16), jnp.float32),
            pltpu.VMEM((npw, 16), jnp.float32),
            pltpu.VMEM_SHARED((npad, 16), jnp.float32),
        ],
    )
    def deg(dst_hbm, out_hbm, idx_d, ones, zeros, acc):
        c = lax.axis_index("c")
        s = lax.axis_index("s")

        def fill_ones(i, carry):
            ones[i] = jnp.ones((16,), jnp.float32)
            return carry

        lax.fori_loop(0, _CHUNK, fill_ones, 0)

        def fill_zeros(i, carry):
            zeros[i] = jnp.zeros((16,), jnp.float32)
            return carry

        lax.fori_loop(0, npw, fill_zeros, 0)

        row0 = s * npw
        pltpu.sync_copy(zeros, acc.at[pl.ds(row0, npw)])
        plsc.subcore_barrier()

        def body(j, carry):
            base = c * half + s * epw + j * _CHUNK
            pltpu.sync_copy(dst_hbm.at[pl.ds(base, _CHUNK)], idx_d)
            pltpu.sync_copy(ones, acc.at[idx_d], add=True)
            return carry

        lax.fori_loop(0, iters, body, 0)
        plsc.subcore_barrier()
        pltpu.sync_copy(acc.at[pl.ds(row0, npw)],
                        out_hbm.at[pl.ds(c * npad + row0, npw)])

    return deg


def _make_agg(npad, e, dh):
    epw = e // _NSUB
    iters = epw // _CHUNK
    npw = npad // _NSUB
    mesh = plsc.VectorSubcoreMesh(core_axis_name="c", subcore_axis_name="s")

    @functools.partial(
        pl.kernel,
        mesh=mesh,
        compiler_params=pltpu.CompilerParams(use_tc_tiling_on_sc=False),
        out_type=jax.ShapeDtypeStruct((_NCORE * npad, dh), jnp.float32),
        scratch_types=[
            pltpu.VMEM((_CHUNK,), jnp.int32),
            pltpu.VMEM((_CHUNK,), jnp.int32),
            pltpu.VMEM((_CHUNK, dh), jnp.float32),
            pltpu.VMEM_SHARED((npad, dh), jnp.float32),
            pltpu.SemaphoreType.DMA,
        ],
    )
    def agg(table_hbm, src2_hbm, dst_hbm, out_hbm, idx_s, idx_d, rows, acc,
            sem):
        c = lax.axis_index("c")
        s = lax.axis_index("s")
        row0 = s * npw
        pltpu.sync_copy(table_hbm.at[pl.ds(c * npad + row0, npw)],
                        acc.at[pl.ds(row0, npw)])
        plsc.subcore_barrier()

        def body(j, carry):
            base = s * epw + j * _CHUNK
            pltpu.sync_copy(src2_hbm.at[pl.ds(c * e + base, _CHUNK)], idx_s)
            pltpu.sync_copy(dst_hbm.at[pl.ds(base, _CHUNK)], idx_d)
            pltpu.async_copy(table_hbm.at[idx_s], rows, sem).wait()
            pltpu.sync_copy(rows, acc.at[idx_d], add=True)
            return carry

        lax.fori_loop(0, iters, body, 0)
        plsc.subcore_barrier()
        pltpu.sync_copy(acc.at[pl.ds(row0, npw)],
                        out_hbm.at[pl.ds(c * npad + row0, npw)])

    return agg


def _dvec(cnt_block):
    counts = (jnp.sum(cnt_block[0], axis=-1)
              + jnp.sum(cnt_block[1], axis=-1)) * (1.0 / 16.0)
    return lax.rsqrt(1.0 + counts)[:, None]


def _tc_scale1(x, w1, cnt, npad, block_rows=2000):
    n, din = x.shape
    dh = w1.shape[1]
    dh2 = dh // 2

    def body(x_ref, w_ref, cnt_ref, out_ref):
        d = _dvec(cnt_ref[...])
        h = jnp.dot(x_ref[...], w_ref[...],
                    preferred_element_type=jnp.float32) * d
        out_ref[0, :, :] = h[:, :dh2]
        out_ref[1, :, :] = h[:, dh2:]

    return pl.pallas_call(
        body,
        grid=(n // block_rows,),
        in_specs=[
            pl.BlockSpec((block_rows, din), lambda i: (i, 0)),
            pl.BlockSpec((din, dh), lambda i: (0, 0)),
            pl.BlockSpec((2, block_rows, 16), lambda i: (0, i, 0)),
        ],
        out_specs=pl.BlockSpec((2, block_rows, dh2), lambda i: (0, i, 0)),
        out_shape=jax.ShapeDtypeStruct((2, npad, dh2), jnp.float32),
    )(x, w1, cnt)


def _tc_mid(acc1, cnt, b1, w2, n, block_rows=2000):
    npad = acc1.shape[1]
    dh2 = acc1.shape[2]
    dout = w2.shape[1]
    do2 = dout // 2

    def body(a_ref, cnt_ref, b_ref, w_ref, out_ref):
        d = _dvec(cnt_ref[...])
        a = a_ref[...]
        full = jnp.concatenate([a[0], a[1]], axis=-1)
        h1 = jnp.maximum(full * d + b_ref[...], 0.0)
        h2 = jnp.dot(h1, w_ref[...], preferred_element_type=jnp.float32) * d
        out_ref[0, :, :] = h2[:, :do2]
        out_ref[1, :, :] = h2[:, do2:]

    return pl.pallas_call(
        body,
        grid=(n // block_rows,),
        in_specs=[
            pl.BlockSpec((2, block_rows, dh2), lambda i: (0, i, 0)),
            pl.BlockSpec((2, block_rows, 16), lambda i: (0, i, 0)),
            pl.BlockSpec((1, 2 * dh2), lambda i: (0, 0)),
            pl.BlockSpec((2 * dh2, dout), lambda i: (0, 0)),
        ],
        out_specs=pl.BlockSpec((2, block_rows, do2), lambda i: (0, i, 0)),
        out_shape=jax.ShapeDtypeStruct((2, npad, do2), jnp.float32),
    )(acc1, cnt, b1, w2)


def _tc_final(acc2, cnt, b2, n, block_rows=2000):
    do2 = acc2.shape[2]

    def body(a_ref, cnt_ref, b_ref, out_ref):
        d = _dvec(cnt_ref[...])
        a = a_ref[...]
        full = jnp.concatenate([a[0], a[1]], axis=-1)
        out_ref[...] = full * d + b_ref[...]

    return pl.pallas_call(
        body,
        grid=(n // block_rows,),
        in_specs=[
            pl.BlockSpec((2, block_rows, do2), lambda i: (0, i, 0)),
            pl.BlockSpec((2, block_rows, 16), lambda i: (0, i, 0)),
            pl.BlockSpec((1, 2 * do2), lambda i: (0, 0)),
        ],
        out_specs=pl.BlockSpec((block_rows, 2 * do2), lambda i: (i, 0)),
        out_shape=jax.ShapeDtypeStruct((n, 2 * do2), jnp.float32),
    )(acc2, cnt, b2)


def kernel(x, edge_index, W1, b1, W2, b2):
    n, _ = x.shape
    e = edge_index.shape[1]
    dh = W1.shape[1]
    dout = W2.shape[1]
    npad = ((n + 127) // 128) * 128

    src = edge_index[0]
    dst = edge_index[1]
    src2 = jnp.concatenate([src, src + npad])

    cnt = _make_deg(npad, e)(dst).reshape(2, npad, 16)

    hs1 = _tc_scale1(x, W1, cnt, npad)
    agg1 = _make_agg(npad, e, dh // 2)(hs1.reshape(2 * npad, dh // 2),
                                       src2, dst)
    hs2 = _tc_mid(agg1.reshape(2, npad, dh // 2), cnt,
                  b1.reshape(1, dh), W2, n)
    agg2 = _make_agg(npad, e, dout // 2)(hs2.reshape(2 * npad, dout // 2),
                                         src2, dst)
    z = _tc_final(agg2.reshape(2, npad, dout // 2), cnt,
                  b2.reshape(1, dout), n)
    return z
